# baseline (speedup 1.0000x reference)
import functools

import jax
import jax.numpy as jnp
from jax import lax
from jax.experimental import pallas as pl
from jax.experimental.pallas import tpu as pltpu
from jax.experimental.pallas import tpu_sc as plsc

NC = 2
NS = 16
NW = NC * NS
NBLK = 16
RU = 4


def kernel(phone, table):
    B, S = phone.shape
    V, D = table.shape
    per_w = B // NW
    n_blk = per_w // NBLK
    idx3 = phone.reshape(NW, per_w, S)
    table_p = jnp.pad(table, ((0, 0), (0, 128 - D)))

    mesh = plsc.VectorSubcoreMesh(core_axis_name="c", subcore_axis_name="s")

    @functools.partial(
        pl.kernel,
        mesh=mesh,
        out_type=jax.ShapeDtypeStruct((B, S, D), jnp.float32),
        scratch_types=[
            pltpu.VMEM((2, NBLK, S), jnp.int32),
            pltpu.VMEM((2, S, 128), jnp.float32),
            pltpu.VMEM((2, S, D), jnp.float32),
            pltpu.SemaphoreType.DMA((2,)),
            pltpu.SemaphoreType.DMA((2,)),
            pltpu.SemaphoreType.DMA((2,)),
        ],
    )
    def emb(idx_hbm, table_hbm, out_hbm, ibuf, rows_v, cbuf, isem, gsem, ssem):
        wid = lax.axis_index("s") * NC + lax.axis_index("c")
        base = wid * per_w

        def idx_block(m):
            bm = m % 2
            return (
                idx_hbm.at[wid, pl.ds(m * NBLK, NBLK)],
                ibuf.at[bm],
                isem.at[bm],
            )

        def gather_parts(m, jj, b):
            bm = m % 2
            yield (
                table_hbm.at[ibuf.at[bm, jj, pl.ds(0, 128)]],
                rows_v.at[b, pl.ds(0, 128)],
            )
            yield (
                table_hbm.at[ibuf.at[bm, jj, pl.ds(128, S - 128)]],
                rows_v.at[b, pl.ds(128, S - 128)],
            )

        def fire_gather(m, jj, b):
            for src, dst in gather_parts(m, jj, b):
                pltpu.async_copy(src, dst, gsem.at[b])

        def wait_gather(m, jj, b):
            for src, dst in gather_parts(m, jj, b):
                pltpu.make_async_copy(src, dst, gsem.at[b]).wait()

        pltpu.async_copy(*idx_block(0))
        for m in range(n_blk):
            if m + 1 < n_blk:
                pltpu.async_copy(*idx_block(m + 1))
            pltpu.make_async_copy(*idx_block(m)).wait()
            fire_gather(m, 0, (m * NBLK) % 2)

            def body(jj, carry):
                j = m * NBLK + jj
                b = lax.rem(j, 2)

                @pl.when(jj + 1 < NBLK)
                def _():
                    fire_gather(m, jj + 1, lax.rem(j + 1, 2))

                wait_gather(m, jj, b)

                @pl.when(j >= 2)
                def _():
                    pltpu.make_async_copy(
                        cbuf.at[b], out_hbm.at[base + j - 2], ssem.at[b]
                    ).wait()

                def compact(i, c):
                    for r4 in range(RU):
                        r = i * RU + r4
                        for g in range(D // 16):
                            cbuf[b, r, pl.ds(g * 16, 16)] = rows_v[
                                b, r, pl.ds(g * 16, 16)
                            ]
                    return c

                lax.fori_loop(0, S // RU, compact, 0)
                pltpu.async_copy(cbuf.at[b], out_hbm.at[base + j], ssem.at[b])
                return carry

            lax.fori_loop(0, NBLK, body, 0)

        for i in range(2):
            j = per_w - 2 + i
            pltpu.make_async_copy(
                cbuf.at[j % 2], out_hbm.at[base + j], ssem.at[j % 2]
            ).wait()

    return emb(idx3, table_p)


# speedup vs baseline: 1.5999x; 1.5999x over previous
import functools

import numpy as np

import jax
import jax.numpy as jnp
from jax import lax
from jax.experimental import pallas as pl
from jax.experimental.pallas import tpu as pltpu
from jax.experimental.pallas import tpu_sc as plsc

NC = 2
NS = 16
NW = NC * NS
NBLK = 32
NBS = 3
L = 16


def kernel(phone, table):
    B, S = phone.shape
    V, D = table.shape
    per_w = B // NW
    n_blk = per_w // NBLK
    n_full = S // L
    tail = S - n_full * L
    idx3 = phone.reshape(NW, per_w, S)
    table_p = jnp.pad(table, ((0, 0), (0, 128 - D)))

    mesh = plsc.VectorSubcoreMesh(core_axis_name="c", subcore_axis_name="s")

    @functools.partial(
        pl.kernel,
        mesh=mesh,
        out_type=jax.ShapeDtypeStruct((B, S, D), jnp.float32),
        compiler_params=pltpu.CompilerParams(needs_layout_passes=False),
        scratch_types=[
            pltpu.VMEM((V, 128), jnp.float32),
            pltpu.VMEM((2, NBLK, S), jnp.int32),
            pltpu.VMEM((NBS, S, D), jnp.float32),
            pltpu.SemaphoreType.DMA((2,)),
            pltpu.SemaphoreType.DMA((NBS,)),
        ],
    )
    def emb(idx_hbm, table_hbm, out_hbm, tbl_v, ibuf, cbuf, isem, ssem):
        wid = lax.axis_index("s") * NC + lax.axis_index("c")
        base = wid * per_w

        def idx_block(m):
            bm = m % 2
            return (
                idx_hbm.at[wid, pl.ds(m * NBLK, NBLK)],
                ibuf.at[bm],
                isem.at[bm],
            )

        pltpu.async_copy(*idx_block(0))
        pltpu.sync_copy(table_hbm, tbl_v)
        iota = lax.iota(jnp.int32, L)

        for m in range(n_blk):
            if m + 1 < n_blk:
                pltpu.async_copy(*idx_block(m + 1))
            pltpu.make_async_copy(*idx_block(m)).wait()
            bm = m % 2

            def body(jj, carry):
                j = m * NBLK + jj
                bs = lax.rem(j, NBS)

                @pl.when(j >= NBS)
                def _():
                    pltpu.make_async_copy(
                        cbuf.at[bs], out_hbm.at[base + j - NBS], ssem.at[bs]
                    ).wait()

                slab = cbuf.at[bs]

                def group(start):
                    v_idx = ibuf[bm, jj, pl.ds(start, L)]
                    rows = start + iota
                    for d0 in range(D):
                        col = d0 + iota
                        col = jnp.where(col < D, col, col - D)
                        val = plsc.load_gather(tbl_v, [v_idx, col])
                        plsc.store_scatter(slab, [rows, col], val)

                @plsc.parallel_loop(0, n_full, 1, unroll=2)
                def _(g):
                    group(g * L)
                if tail:
                    group(S - L)

                pltpu.async_copy(slab, out_hbm.at[base + j], ssem.at[bs])
                return carry

            lax.fori_loop(0, NBLK, body, 0)

        for i in range(NBS):
            j = per_w - NBS + i
            pltpu.make_async_copy(
                cbuf.at[j % NBS], out_hbm.at[base + j], ssem.at[j % NBS]
            ).wait()

    return emb(idx3, table_p)


# speedup vs baseline: 1.9505x; 1.2191x over previous
import functools

import numpy as np

import jax
import jax.numpy as jnp
from jax import lax
from jax.experimental import pallas as pl
from jax.experimental.pallas import tpu as pltpu
from jax.experimental.pallas import tpu_sc as plsc

NC = 2
NS = 16
NW = NC * NS
NBLK = 32
NBS = 3
L = 16


def kernel(phone, table):
    B, S = phone.shape
    V, D = table.shape
    per_w = B // NW
    n_blk = per_w // NBLK
    n_full = S // L
    tail = S - n_full * L
    idx3 = phone.reshape(NW, per_w, S)
    table_p = jnp.pad(table, ((0, 0), (0, 128 - D)))

    mesh = plsc.VectorSubcoreMesh(core_axis_name="c", subcore_axis_name="s")

    @functools.partial(
        pl.kernel,
        mesh=mesh,
        out_type=jax.ShapeDtypeStruct((B, S, D), jnp.float32),
        compiler_params=pltpu.CompilerParams(needs_layout_passes=False),
        scratch_types=[
            pltpu.VMEM((V, 128), jnp.float32),
            pltpu.VMEM((2, NBLK, S), jnp.int32),
            pltpu.VMEM((NBS, S, D), jnp.float32),
            pltpu.SemaphoreType.DMA((2,)),
            pltpu.SemaphoreType.DMA((NBS,)),
        ],
    )
    def emb(idx_hbm, table_hbm, out_hbm, tbl_v, ibuf, cbuf, isem, ssem):
        wid = lax.axis_index("s") * NC + lax.axis_index("c")
        base = wid * per_w

        def idx_block(m):
            bm = m % 2
            return (
                idx_hbm.at[wid, pl.ds(m * NBLK, NBLK)],
                ibuf.at[bm],
                isem.at[bm],
            )

        pltpu.async_copy(*idx_block(0))
        pltpu.sync_copy(table_hbm, tbl_v)
        iota = lax.iota(jnp.int32, L)

        for m in range(n_blk):
            if m + 1 < n_blk:
                pltpu.async_copy(*idx_block(m + 1))
            pltpu.make_async_copy(*idx_block(m)).wait()
            bm = m % 2

            def body(jj, carry):
                j = m * NBLK + jj
                bs = lax.rem(j, NBS)

                @pl.when(j >= NBS)
                def _():
                    pltpu.make_async_copy(
                        cbuf.at[bs], out_hbm.at[base + j - NBS], ssem.at[bs]
                    ).wait()

                slab = cbuf.at[bs]

                def group(start):
                    v_idx = ibuf[bm, jj, pl.ds(start, L)]
                    rows = start + iota
                    @plsc.parallel_loop(0, D, 1, unroll=8)
                    def _(d0):
                        col = d0 + iota
                        col = jnp.where(col < D, col, col - D)
                        val = plsc.load_gather(tbl_v, [v_idx, col])
                        plsc.store_scatter(slab, [rows, col], val)

                @plsc.parallel_loop(0, n_full, 1, unroll=2)
                def _(g):
                    group(g * L)
                if tail:
                    group(S - L)

                pltpu.async_copy(slab, out_hbm.at[base + j], ssem.at[bs])
                return carry

            lax.fori_loop(0, NBLK, body, 0)

        for i in range(NBS):
            j = per_w - NBS + i
            pltpu.make_async_copy(
                cbuf.at[j % NBS], out_hbm.at[base + j], ssem.at[j % NBS]
            ).wait()

    return emb(idx3, table_p)


# speedup vs baseline: 1.9608x; 1.0053x over previous
import functools

import numpy as np

import jax
import jax.numpy as jnp
from jax import lax
from jax.experimental import pallas as pl
from jax.experimental.pallas import tpu as pltpu
from jax.experimental.pallas import tpu_sc as plsc

NC = 2
NS = 16
NW = NC * NS
NBLK = 32
NBS = 3
L = 16


def kernel(phone, table):
    B, S = phone.shape
    V, D = table.shape
    per_w = B // NW
    n_blk = per_w // NBLK
    n_full = S // L
    tail = S - n_full * L
    idx3 = phone.reshape(NW, per_w, S)
    table_p = jnp.pad(table, ((0, 0), (0, 128 - D)))

    mesh = plsc.VectorSubcoreMesh(core_axis_name="c", subcore_axis_name="s")

    @functools.partial(
        pl.kernel,
        mesh=mesh,
        out_type=jax.ShapeDtypeStruct((B, S, D), jnp.float32),
        compiler_params=pltpu.CompilerParams(needs_layout_passes=False),
        scratch_types=[
            pltpu.VMEM((V, 128), jnp.float32),
            pltpu.VMEM((2, NBLK, S), jnp.int32),
            pltpu.VMEM((NBS, S, D), jnp.float32),
            pltpu.SemaphoreType.DMA((2,)),
            pltpu.SemaphoreType.DMA((NBS,)),
        ],
    )
    def emb(idx_hbm, table_hbm, out_hbm, tbl_v, ibuf, cbuf, isem, ssem):
        wid = lax.axis_index("s") * NC + lax.axis_index("c")
        base = wid * per_w

        def idx_block(m):
            bm = m % 2
            return (
                idx_hbm.at[wid, pl.ds(m * NBLK, NBLK)],
                ibuf.at[bm],
                isem.at[bm],
            )

        pltpu.async_copy(*idx_block(0))
        pltpu.sync_copy(table_hbm, tbl_v)
        iota = lax.iota(jnp.int32, L)

        for m in range(n_blk):
            if m + 1 < n_blk:
                pltpu.async_copy(*idx_block(m + 1))
            pltpu.make_async_copy(*idx_block(m)).wait()
            bm = m % 2

            def body(jj, carry):
                j = m * NBLK + jj
                bs = lax.rem(j, NBS)

                @pl.when(j >= NBS)
                def _():
                    pltpu.make_async_copy(
                        cbuf.at[bs], out_hbm.at[base + j - NBS], ssem.at[bs]
                    ).wait()

                slab = cbuf.at[bs]

                def group(start):
                    v_idx = ibuf[bm, jj, pl.ds(start, L)]
                    rows = start + iota
                    @plsc.parallel_loop(0, D, 1, unroll=16)
                    def _(d0):
                        col = d0 + iota
                        col = jnp.where(col < D, col, col - D)
                        val = plsc.load_gather(tbl_v, [v_idx, col])
                        plsc.store_scatter(slab, [rows, col], val)

                @plsc.parallel_loop(0, n_full, 1, unroll=2)
                def _(g):
                    group(g * L)
                if tail:
                    group(S - L)

                pltpu.async_copy(slab, out_hbm.at[base + j], ssem.at[bs])
                return carry

            lax.fori_loop(0, NBLK, body, 0)

        for i in range(NBS):
            j = per_w - NBS + i
            pltpu.make_async_copy(
                cbuf.at[j % NBS], out_hbm.at[base + j], ssem.at[j % NBS]
            ).wait()

    return emb(idx3, table_p)


# speedup vs baseline: 1.9643x; 1.0018x over previous
import functools

import numpy as np

import jax
import jax.numpy as jnp
from jax import lax
from jax.experimental import pallas as pl
from jax.experimental.pallas import tpu as pltpu
from jax.experimental.pallas import tpu_sc as plsc

NC = 2
NS = 16
NW = NC * NS
NBLK = 32
NBS = 3
L = 16


def kernel(phone, table):
    B, S = phone.shape
    V, D = table.shape
    per_w = B // NW
    n_blk = per_w // NBLK
    n_full = S // L
    tail = S - n_full * L
    idx3 = phone.reshape(NW, per_w, S)
    table_p = jnp.pad(table, ((0, 0), (0, 128 - D)))

    mesh = plsc.VectorSubcoreMesh(core_axis_name="c", subcore_axis_name="s")

    @functools.partial(
        pl.kernel,
        mesh=mesh,
        out_type=jax.ShapeDtypeStruct((B, S, D), jnp.float32),
        compiler_params=pltpu.CompilerParams(needs_layout_passes=False),
        scratch_types=[
            pltpu.VMEM((V, 128), jnp.float32),
            pltpu.VMEM((2, NBLK, S), jnp.int32),
            pltpu.VMEM((NBS, S, D), jnp.float32),
            pltpu.SemaphoreType.DMA((2,)),
            pltpu.SemaphoreType.DMA((NBS,)),
        ],
    )
    def emb(idx_hbm, table_hbm, out_hbm, tbl_v, ibuf, cbuf, isem, ssem):
        wid = lax.axis_index("s") * NC + lax.axis_index("c")
        base = wid * per_w

        def idx_block(m):
            bm = m % 2
            return (
                idx_hbm.at[wid, pl.ds(m * NBLK, NBLK)],
                ibuf.at[bm],
                isem.at[bm],
            )

        pltpu.async_copy(*idx_block(0))
        pltpu.sync_copy(table_hbm, tbl_v)
        iota = lax.iota(jnp.int32, L)

        for m in range(n_blk):
            if m + 1 < n_blk:
                pltpu.async_copy(*idx_block(m + 1))
            pltpu.make_async_copy(*idx_block(m)).wait()
            bm = m % 2

            def body(jj, carry):
                j = m * NBLK + jj
                bs = lax.rem(j, NBS)

                @pl.when(j >= NBS)
                def _():
                    pltpu.make_async_copy(
                        cbuf.at[bs], out_hbm.at[base + j - NBS], ssem.at[bs]
                    ).wait()

                slab = cbuf.at[bs]

                def group(start):
                    v_idx = ibuf[bm, jj, pl.ds(start, L)]
                    rows = start + iota
                    @plsc.parallel_loop(0, D, 1, unroll=16)
                    def _(d0):
                        col = d0 + iota
                        col = jnp.where(col < D, col, col - D)
                        val = plsc.load_gather(tbl_v, [v_idx, col])
                        plsc.store_scatter(slab, [rows, col], val)

                @plsc.parallel_loop(0, n_full, 1, unroll=4)
                def _(g):
                    group(g * L)
                if tail:
                    group(S - L)

                pltpu.async_copy(slab, out_hbm.at[base + j], ssem.at[bs])
                return carry

            lax.fori_loop(0, NBLK, body, 0)

        for i in range(NBS):
            j = per_w - NBS + i
            pltpu.make_async_copy(
                cbuf.at[j % NBS], out_hbm.at[base + j], ssem.at[j % NBS]
            ).wait()

    return emb(idx3, table_p)


# speedup vs baseline: 1.9651x; 1.0004x over previous
import functools

import numpy as np

import jax
import jax.numpy as jnp
from jax import lax
from jax.experimental import pallas as pl
from jax.experimental.pallas import tpu as pltpu
from jax.experimental.pallas import tpu_sc as plsc

NC = 2
NS = 16
NW = NC * NS
NBLK = 32
NBS = 3
L = 16


def kernel(phone, table):
    B, S = phone.shape
    V, D = table.shape
    per_w = B // NW
    n_blk = per_w // NBLK
    n_full = S // L
    tail = S - n_full * L
    idx3 = phone.reshape(NW, per_w, S)
    table_p = jnp.pad(table, ((0, 0), (0, 128 - D)))

    mesh = plsc.VectorSubcoreMesh(core_axis_name="c", subcore_axis_name="s")

    @functools.partial(
        pl.kernel,
        mesh=mesh,
        out_type=jax.ShapeDtypeStruct((B, S, D), jnp.float32),
        compiler_params=pltpu.CompilerParams(needs_layout_passes=False, skip_device_barrier=True, disable_bounds_checks=True),
        scratch_types=[
            pltpu.VMEM((V, 128), jnp.float32),
            pltpu.VMEM((2, NBLK, S), jnp.int32),
            pltpu.VMEM((NBS, S, D), jnp.float32),
            pltpu.SemaphoreType.DMA((2,)),
            pltpu.SemaphoreType.DMA((NBS,)),
        ],
    )
    def emb(idx_hbm, table_hbm, out_hbm, tbl_v, ibuf, cbuf, isem, ssem):
        wid = lax.axis_index("s") * NC + lax.axis_index("c")
        base = wid * per_w

        def idx_block(m):
            bm = m % 2
            return (
                idx_hbm.at[wid, pl.ds(m * NBLK, NBLK)],
                ibuf.at[bm],
                isem.at[bm],
            )

        pltpu.async_copy(*idx_block(0))
        pltpu.sync_copy(table_hbm, tbl_v)
        iota = lax.iota(jnp.int32, L)

        for m in range(n_blk):
            if m + 1 < n_blk:
                pltpu.async_copy(*idx_block(m + 1))
            pltpu.make_async_copy(*idx_block(m)).wait()
            bm = m % 2

            def body(jj, carry):
                j = m * NBLK + jj
                bs = lax.rem(j, NBS)

                @pl.when(j >= NBS)
                def _():
                    pltpu.make_async_copy(
                        cbuf.at[bs], out_hbm.at[base + j - NBS], ssem.at[bs]
                    ).wait()

                slab = cbuf.at[bs]

                def group(start):
                    v_idx = ibuf[bm, jj, pl.ds(start, L)]
                    rows = start + iota
                    @plsc.parallel_loop(0, D, 1, unroll=16)
                    def _(d0):
                        col = d0 + iota
                        col = jnp.where(col < D, col, col - D)
                        val = plsc.load_gather(tbl_v, [v_idx, col])
                        plsc.store_scatter(slab, [rows, col], val)

                @plsc.parallel_loop(0, n_full, 1, unroll=4)
                def _(g):
                    group(g * L)
                if tail:
                    group(S - L)

                pltpu.async_copy(slab, out_hbm.at[base + j], ssem.at[bs])
                return carry

            lax.fori_loop(0, NBLK, body, 0)

        for i in range(NBS):
            j = per_w - NBS + i
            pltpu.make_async_copy(
                cbuf.at[j % NBS], out_hbm.at[base + j], ssem.at[j % NBS]
            ).wait()

    return emb(idx3, table_p)


# speedup vs baseline: 2.0754x; 1.0561x over previous
import functools

import numpy as np

import jax
import jax.numpy as jnp
from jax import lax
from jax.experimental import pallas as pl
from jax.experimental.pallas import tpu as pltpu
from jax.experimental.pallas import tpu_sc as plsc

NC = 2
NS = 16
NW = NC * NS
NBLK = 32
NBS = 3
L = 16


def kernel(phone, table):
    B, S = phone.shape
    V, D = table.shape
    per_w = B // NW
    n_blk = per_w // NBLK
    n_full = S // L
    tail = S - n_full * L
    idx3 = phone.reshape(NW, per_w, S)
    table_p = jnp.pad(table, ((0, 0), (0, 128 - D))).reshape(-1)

    mesh = plsc.VectorSubcoreMesh(core_axis_name="c", subcore_axis_name="s")

    @functools.partial(
        pl.kernel,
        mesh=mesh,
        out_type=jax.ShapeDtypeStruct((B, S, D), jnp.float32),
        compiler_params=pltpu.CompilerParams(needs_layout_passes=False, skip_device_barrier=True, disable_bounds_checks=True),
        scratch_types=[
            pltpu.VMEM((V * 128,), jnp.float32),
            pltpu.VMEM((2, NBLK, S), jnp.int32),
            pltpu.VMEM((NBS, S, D), jnp.float32),
            pltpu.SemaphoreType.DMA((2,)),
            pltpu.SemaphoreType.DMA((NBS,)),
        ],
    )
    def emb(idx_hbm, table_hbm, out_hbm, tbl_v, ibuf, cbuf, isem, ssem):
        wid = lax.axis_index("s") * NC + lax.axis_index("c")
        base = wid * per_w

        def idx_block(m):
            bm = m % 2
            return (
                idx_hbm.at[wid, pl.ds(m * NBLK, NBLK)],
                ibuf.at[bm],
                isem.at[bm],
            )

        pltpu.async_copy(*idx_block(0))
        pltpu.sync_copy(table_hbm, tbl_v)
        iota = lax.iota(jnp.int32, L)

        for m in range(n_blk):
            if m + 1 < n_blk:
                pltpu.async_copy(*idx_block(m + 1))
            pltpu.make_async_copy(*idx_block(m)).wait()
            bm = m % 2

            def body(jj, carry):
                j = m * NBLK + jj
                bs = lax.rem(j, NBS)

                @pl.when(j >= NBS)
                def _():
                    pltpu.make_async_copy(
                        cbuf.at[bs], out_hbm.at[base + j - NBS], ssem.at[bs]
                    ).wait()

                slab = cbuf.at[bs]

                def group(start):
                    v_idx = ibuf[bm, jj, pl.ds(start, L)]
                    v_base = v_idx * 128
                    rows = start + iota
                    @plsc.parallel_loop(0, D, 1, unroll=16)
                    def _(d0):
                        col = d0 + iota
                        col = jnp.where(col < D, col, col - D)
                        val = plsc.load_gather(tbl_v, [v_base + col])
                        plsc.store_scatter(slab, [rows, col], val)

                @plsc.parallel_loop(0, n_full, 1, unroll=4)
                def _(g):
                    group(g * L)
                if tail:
                    group(S - L)

                pltpu.async_copy(slab, out_hbm.at[base + j], ssem.at[bs])
                return carry

            lax.fori_loop(0, NBLK, body, 0)

        for i in range(NBS):
            j = per_w - NBS + i
            pltpu.make_async_copy(
                cbuf.at[j % NBS], out_hbm.at[base + j], ssem.at[j % NBS]
            ).wait()

    return emb(idx3, table_p)
